# baseline (speedup 1.0000x reference)
import jax
import jax.numpy as jnp
from jax.experimental import pallas as pl
from jax.experimental.pallas import tpu as pltpu

_V = 100000
_EPS = 0.1
_CONF = 1.0 - _EPS
_SMOOTH = _EPS / (_V - 2)
_IGNORE = 0

_LANES = 128
_VA = _V // _LANES * _LANES
_CMAX = (_V - 160) // _LANES * _LANES
_TAIL = _CMAX + _LANES
_BR = 32


def _loss_kernel(t_sm, x_ref, pred_any, t_ref, out_ref, sliver_ref, sem):
    bi = pl.program_id(0)
    base = bi * _BR

    def _sliver_copy(local):
        r = base + local
        c = jnp.minimum((t_sm[r] // _LANES) * _LANES, _CMAX)
        return pltpu.make_async_copy(
            pred_any.at[r, pl.ds(c, _LANES)], sliver_ref.at[local], sem)

    def _issue(local, carry):
        _sliver_copy(local).start()
        return carry

    jax.lax.fori_loop(0, _BR, _issue, 0)

    x = x_ref[...]
    xa = x[:, :_VA]
    xt = x[:, _VA:_V]
    m = jnp.maximum(jnp.max(xa, axis=1, keepdims=True),
                    jnp.max(xt, axis=1, keepdims=True))
    rowsum = (jnp.sum(xa, axis=1, keepdims=True)
              + jnp.sum(xt, axis=1, keepdims=True))
    es = (jnp.sum(jnp.exp(xa - m), axis=1, keepdims=True)
          + jnp.sum(jnp.exp(xt - m), axis=1, keepdims=True))
    p0 = x[:, 0:1]

    t = t_ref[...]
    tail_ids = _VA + jax.lax.broadcasted_iota(jnp.int32, (_BR, _V - _VA), 1)
    pt_tail = jnp.sum(jnp.where(tail_ids == t, xt, 0.0), axis=1,
                      keepdims=True)

    def _wait(local, carry):
        _sliver_copy(local).wait()
        return carry

    jax.lax.fori_loop(0, _BR, _wait, 0)
    g = sliver_ref[...]
    c_vec = jnp.minimum((t // _LANES) * _LANES, _CMAX)
    lane = t - c_vec
    lane_ids = jax.lax.broadcasted_iota(jnp.int32, (_BR, _LANES), 1)
    pt_sliver = jnp.sum(jnp.where(lane_ids == lane, g, 0.0), axis=1,
                        keepdims=True)
    pt = jnp.where(t >= _TAIL, pt_tail, pt_sliver)

    lse = m + jnp.log(es)
    logp_t = pt - lse
    logp_0 = p0 - lse
    s_row = rowsum - jnp.float32(_V) * lse
    contrib = _SMOOTH * (s_row - logp_t - logp_0) + _CONF * logp_t
    rmask = t != _IGNORE
    contrib = jnp.where(rmask, contrib, 0.0)
    csum = jnp.sum(contrib).reshape(1, 1)
    nv = jnp.sum(rmask.astype(jnp.float32)).reshape(1, 1)
    out_ref[...] = jnp.concatenate([csum, nv], axis=1).reshape(1, 1, 2)


def _combine_kernel(p_ref, out_ref):
    p = p_ref[...].reshape(-1, 2)
    csum = jnp.sum(p[:, 0:1])
    nv = jnp.sum(p[:, 1:2])
    out_ref[...] = (-csum / jnp.maximum(nv, 1.0)).reshape(1, 1)


def kernel(pred, target):
    pred2 = pred.reshape(-1, pred.shape[-1])
    n = pred2.shape[0]
    t = target.reshape(n).astype(jnp.int32)
    nb = n // _BR

    grid_spec = pltpu.PrefetchScalarGridSpec(
        num_scalar_prefetch=1,
        grid=(nb,),
        in_specs=[
            pl.BlockSpec((_BR, _V), lambda b, t_sm: (b, 0)),
            pl.BlockSpec(memory_space=pltpu.MemorySpace.HBM),
            pl.BlockSpec((_BR, 1), lambda b, t_sm: (b, 0)),
        ],
        out_specs=pl.BlockSpec((1, 1, 2), lambda b, t_sm: (b, 0, 0)),
        scratch_shapes=[
            pltpu.VMEM((_BR, _LANES), jnp.float32),
            pltpu.SemaphoreType.DMA,
        ],
    )
    partials = pl.pallas_call(
        _loss_kernel,
        grid_spec=grid_spec,
        out_shape=jax.ShapeDtypeStruct((nb, 1, 2), jnp.float32),
        compiler_params=pltpu.CompilerParams(
            dimension_semantics=("parallel",)),
    )(t, pred2, pred2, t.reshape(n, 1))

    out = pl.pallas_call(
        _combine_kernel,
        out_shape=jax.ShapeDtypeStruct((1, 1), jnp.float32),
    )(partials)
    return out[0, 0]
